# baseline (speedup 1.0000x reference)
import functools

import jax
import jax.numpy as jnp
from jax import lax
from jax.experimental import pallas as pl
from jax.experimental.pallas import tpu as pltpu
from jax.experimental.pallas import tpu_sc as plsc

_C = 128


@functools.cache
def _make_gather(D, B_pad, NC, NS):
    NW = NC * NS
    b_per_w = B_pad // NW
    n_chunks = b_per_w // _C
    mesh = plsc.VectorSubcoreMesh(core_axis_name="c", subcore_axis_name="s")

    @functools.partial(
        pl.kernel,
        mesh=mesh,
        out_type=jax.ShapeDtypeStruct((B_pad, D), jnp.float32),
        scratch_types=[
            pltpu.VMEM((b_per_w,), jnp.int32),
            pltpu.VMEM((_C, D), jnp.float32),
            pltpu.SemaphoreType.DMA,
        ],
    )
    def k(x_hbm, idx_hbm, out_hbm, idx_v, buf, gsem):
        wid = lax.axis_index("s") * NC + lax.axis_index("c")
        pltpu.sync_copy(idx_hbm.at[pl.ds(wid * b_per_w, b_per_w)], idx_v)
        out_row0 = wid * b_per_w
        for j in range(n_chunks):
            pltpu.async_copy(
                x_hbm.at[idx_v.at[pl.ds(j * _C, _C)]], buf, gsem
            ).wait()
            pltpu.sync_copy(buf, out_hbm.at[pl.ds(out_row0 + j * _C, _C)])

    return k


def kernel(x, upsample_inds):
    D = x.shape[1]
    B = upsample_inds.shape[0]
    idx = upsample_inds[:, 0].astype(jnp.int32)
    info = plsc.get_sparse_core_info()
    NC, NS = info.num_cores, info.num_subcores
    span = NC * NS * _C
    B_pad = ((B + span - 1) // span) * span
    idx_pad = jnp.concatenate([idx, jnp.zeros((B_pad - B,), jnp.int32)])
    out = _make_gather(D, B_pad, NC, NS)(x, idx_pad)
    return out[:B]


# speedup vs baseline: 1.1041x; 1.1041x over previous
import functools

import jax
import jax.numpy as jnp
from jax import lax
from jax.experimental import pallas as pl
from jax.experimental.pallas import tpu as pltpu
from jax.experimental.pallas import tpu_sc as plsc

_C = 128


_NB = 7


@functools.cache
def _make_gather(D, B_pad, NC, NS):
    NW = NC * NS
    b_per_w = B_pad // NW
    n_chunks = b_per_w // _C
    mesh = plsc.VectorSubcoreMesh(core_axis_name="c", subcore_axis_name="s")

    @functools.partial(
        pl.kernel,
        mesh=mesh,
        out_type=jax.ShapeDtypeStruct((B_pad, D), jnp.float32),
        scratch_types=[
            pltpu.VMEM((b_per_w,), jnp.int32),
            *[pltpu.VMEM((_C, D), jnp.float32) for _ in range(_NB)],
            *[pltpu.SemaphoreType.DMA for _ in range(2 * _NB)],
        ],
    )
    def k(x_hbm, idx_hbm, out_hbm, idx_v, *rest):
        bufs = rest[:_NB]
        gsem = rest[_NB : 2 * _NB]
        ssem = rest[2 * _NB :]
        wid = lax.axis_index("s") * NC + lax.axis_index("c")
        pltpu.sync_copy(idx_hbm.at[pl.ds(wid * b_per_w, b_per_w)], idx_v)
        out_row0 = wid * b_per_w

        def start_gather(j):
            b = j % _NB
            return pltpu.async_copy(
                x_hbm.at[idx_v.at[pl.ds(j * _C, _C)]], bufs[b], gsem[b]
            )

        def start_store(j):
            b = j % _NB
            return pltpu.async_copy(
                bufs[b], out_hbm.at[pl.ds(out_row0 + j * _C, _C)], ssem[b]
            )

        ghandles = [start_gather(j) for j in range(_NB)]
        shandles = [None] * n_chunks
        for j in range(n_chunks):
            ghandles[j % _NB].wait()
            shandles[j] = start_store(j)
            nxt = j + _NB
            if nxt < n_chunks:
                shandles[j].wait()
                ghandles[nxt % _NB] = start_gather(nxt)
        for j in range(n_chunks - _NB, n_chunks):
            shandles[j].wait()

    return k


def kernel(x, upsample_inds):
    D = x.shape[1]
    B = upsample_inds.shape[0]
    idx = upsample_inds[:, 0].astype(jnp.int32)
    info = plsc.get_sparse_core_info()
    NC, NS = info.num_cores, info.num_subcores
    span = NC * NS * _C
    B_pad = ((B + span - 1) // span) * span
    idx_pad = jnp.concatenate([idx, jnp.zeros((B_pad - B,), jnp.int32)])
    out = _make_gather(D, B_pad, NC, NS)(x, idx_pad)
    return out[:B]


# speedup vs baseline: 3.0057x; 2.7223x over previous
import functools

import jax
import jax.numpy as jnp
from jax import lax
from jax.experimental import pallas as pl
from jax.experimental.pallas import tpu as pltpu
from jax.experimental.pallas import tpu_sc as plsc

_C = 128


_NB = 7


@functools.cache
def _make_gather(D, B_pad, NC, NS):
    NW = NC * NS
    b_per_w = B_pad // NW
    n_chunks = b_per_w // _C
    mesh = plsc.VectorSubcoreMesh(core_axis_name="c", subcore_axis_name="s")

    @functools.partial(
        pl.kernel,
        mesh=mesh,
        out_type=jax.ShapeDtypeStruct((B_pad, D), jnp.float32),
        scratch_types=[
            pltpu.VMEM((b_per_w,), jnp.int32),
            *[pltpu.VMEM((_C, D), jnp.float32) for _ in range(_NB)],
            *[pltpu.SemaphoreType.DMA for _ in range(2 * _NB)],
        ],
    )
    def k(x_hbm, idx_hbm, out_hbm, idx_v, *rest):
        bufs = rest[:_NB]
        gsem = rest[_NB : 2 * _NB]
        ssem = rest[2 * _NB :]
        wid = lax.axis_index("s") * NC + lax.axis_index("c")
        pltpu.sync_copy(idx_hbm.at[pl.ds(wid * b_per_w, b_per_w)], idx_v)
        out_row0 = wid * b_per_w

        def start_gather(j):
            b = j % _NB
            return pltpu.async_copy(
                x_hbm.at[idx_v.at[pl.ds(j * _C, _C)]], bufs[b], gsem[b]
            )

        def start_store(j):
            b = j % _NB
            return pltpu.async_copy(
                bufs[b], out_hbm.at[pl.ds(out_row0 + j * _C, _C)], ssem[b]
            )

        ghandles = [start_gather(j) for j in range(_NB)]
        shandles = [None] * n_chunks
        for j in range(n_chunks):
            ghandles[j % _NB].wait()
            shandles[j] = start_store(j)
            nxt = j + _NB
            if nxt < n_chunks:
                shandles[j].wait()
                ghandles[nxt % _NB] = start_gather(nxt)
        for j in range(n_chunks - _NB, n_chunks):
            shandles[j].wait()

    return k


def kernel(x, upsample_inds):
    D = x.shape[1]
    B = upsample_inds.shape[0]
    idx = upsample_inds[:, 0].astype(jnp.int32)
    info = plsc.get_sparse_core_info()
    NC, NS = info.num_cores, info.num_subcores
    span = NC * NS * _C
    B_pad = ((B + span - 1) // span) * span
    pad = jnp.arange(B_pad - B, dtype=jnp.int32) % x.shape[0]
    idx_pad = jnp.concatenate([idx, pad])
    out = _make_gather(D, B_pad, NC, NS)(x, idx_pad)
    return out[:B]


# speedup vs baseline: 4.8179x; 1.6029x over previous
import functools

import jax
import jax.numpy as jnp
from jax import lax
from jax.experimental import pallas as pl
from jax.experimental.pallas import tpu as pltpu
from jax.experimental.pallas import tpu_sc as plsc

_C = 128
_NB = 7


@functools.cache
def _make_gather(D, B, NC, NS):
    NW = NC * NS
    span = NW * _C
    B_pad = ((B + span - 1) // span) * span
    b_per_w = B_pad // NW
    n_chunks = b_per_w // _C
    mesh = plsc.VectorSubcoreMesh(core_axis_name="c", subcore_axis_name="s")

    @functools.partial(
        pl.kernel,
        mesh=mesh,
        out_type=jax.ShapeDtypeStruct((B, D), jnp.float32),
        scratch_types=[
            pltpu.VMEM((b_per_w,), jnp.int32),
            *[pltpu.VMEM((_C, D), jnp.float32) for _ in range(_NB)],
            *[pltpu.SemaphoreType.DMA for _ in range(2 * _NB)],
        ],
    )
    def k(x_hbm, idx_hbm, out_hbm, idx_v, *rest):
        bufs = rest[:_NB]
        gsem = rest[_NB : 2 * _NB]
        ssem = rest[2 * _NB :]
        wid = lax.axis_index("s") * NC + lax.axis_index("c")
        pltpu.sync_copy(idx_hbm.at[pl.ds(wid * b_per_w, b_per_w)], idx_v)
        base = wid * b_per_w

        def active(j):
            return base + j * _C < B

        def is_norm(j):
            return base + (j + 1) * _C <= B

        def is_bound(j):
            return jnp.logical_and(active(j), jnp.logical_not(is_norm(j)))

        def start_gather(j):
            b = j % _NB
            def _gn():
                pltpu.async_copy(
                    x_hbm.at[idx_v.at[pl.ds(j * _C, _C)]], bufs[b], gsem[b]
                )

            def _gb():
                pltpu.async_copy(
                    x_hbm.at[idx_v.at[pl.ds((B - _C) - base, _C)]],
                    bufs[b],
                    gsem[b],
                )

            pl.when(is_norm(j))(_gn)
            pl.when(is_bound(j))(_gb)

        def wait_gather(j):
            b = j % _NB
            pltpu.make_async_copy(
                x_hbm.at[idx_v.at[pl.ds(0, _C)]], bufs[b], gsem[b]
            ).wait()

        def start_store(j):
            b = j % _NB
            def _sn():
                pltpu.async_copy(
                    bufs[b], out_hbm.at[pl.ds(base + j * _C, _C)], ssem[b]
                )

            def _sb():
                pltpu.async_copy(
                    bufs[b], out_hbm.at[pl.ds(B - _C, _C)], ssem[b]
                )

            pl.when(is_norm(j))(_sn)
            pl.when(is_bound(j))(_sb)

        def wait_store(j):
            b = j % _NB
            pltpu.make_async_copy(
                bufs[b], out_hbm.at[pl.ds(0, _C)], ssem[b]
            ).wait()

        for j in range(_NB):
            start_gather(j)
        for j in range(n_chunks):
            pl.when(active(j))(lambda j=j: wait_gather(j))
            start_store(j)
            nxt = j + _NB
            if nxt < n_chunks:
                pl.when(active(nxt))(lambda j=j: wait_store(j))
                start_gather(nxt)
        for j in range(n_chunks):
            nxt = j + _NB
            if nxt >= n_chunks:
                tail = active(j)
            else:
                tail = jnp.logical_and(active(j), jnp.logical_not(active(nxt)))
            pl.when(tail)(lambda j=j: wait_store(j))

    return k


def kernel(x, upsample_inds):
    D = x.shape[1]
    B = upsample_inds.shape[0]
    idx = upsample_inds[:, 0].astype(jnp.int32)
    info = plsc.get_sparse_core_info()
    NC, NS = info.num_cores, info.num_subcores
    span = NC * NS * _C
    B_pad = ((B + span - 1) // span) * span
    pad = jnp.arange(B_pad - B, dtype=jnp.int32) % x.shape[0]
    idx_pad = jnp.concatenate([idx, pad])
    return _make_gather(D, B, NC, NS)(x, idx_pad)


# speedup vs baseline: 4.8425x; 1.0051x over previous
import functools

import jax
import jax.numpy as jnp
from jax import lax
from jax.experimental import pallas as pl
from jax.experimental.pallas import tpu as pltpu
from jax.experimental.pallas import tpu_sc as plsc

_C = 128
_NB = 7


@functools.cache
def _make_gather(D, B, NC, NS):
    NW = NC * NS
    span = NW * _C
    B_pad = ((B + span - 1) // span) * span
    b_per_w = B_pad // NW
    n_chunks = b_per_w // _C
    mesh = plsc.VectorSubcoreMesh(core_axis_name="c", subcore_axis_name="s")

    @functools.partial(
        pl.kernel,
        mesh=mesh,
        out_type=jax.ShapeDtypeStruct((B, D), jnp.float32),
        scratch_types=[
            pltpu.VMEM((b_per_w,), jnp.int32),
            *[pltpu.VMEM((_C, D), jnp.float32) for _ in range(_NB)],
            *[pltpu.SemaphoreType.DMA for _ in range(2 * _NB)],
        ],
    )
    def k(x_hbm, idx_hbm, out_hbm, idx_v, *rest):
        bufs = rest[:_NB]
        gsem = rest[_NB : 2 * _NB]
        ssem = rest[2 * _NB :]
        wid = lax.axis_index("s") * NC + lax.axis_index("c")
        pltpu.sync_copy(idx_hbm.at[pl.ds(wid * b_per_w, b_per_w)], idx_v)
        base = wid * b_per_w

        def active(j):
            return base + j * _C < B

        def is_norm(j):
            return base + (j + 1) * _C <= B

        def is_bound(j):
            return jnp.logical_and(active(j), jnp.logical_not(is_norm(j)))

        def start_gather(j):
            b = j % _NB
            def _gn():
                pltpu.async_copy(
                    x_hbm.at[idx_v.at[pl.ds(j * _C, _C)]], bufs[b], gsem[b]
                )

            def _gb():
                pltpu.async_copy(
                    x_hbm.at[idx_v.at[pl.ds((B - _C) - base, _C)]],
                    bufs[b],
                    gsem[b],
                )

            pl.when(is_norm(j))(_gn)
            pl.when(is_bound(j))(_gb)

        def wait_gather(j):
            b = j % _NB
            pltpu.make_async_copy(
                x_hbm.at[idx_v.at[pl.ds(0, _C)]], bufs[b], gsem[b]
            ).wait()

        def start_store(j):
            b = j % _NB
            def _sn():
                pltpu.async_copy(
                    bufs[b], out_hbm.at[pl.ds(base + j * _C, _C)], ssem[b]
                )

            def _sb():
                pltpu.async_copy(
                    bufs[b], out_hbm.at[pl.ds(B - _C, _C)], ssem[b]
                )

            pl.when(is_norm(j))(_sn)
            pl.when(is_bound(j))(_sb)

        def wait_store(j):
            b = j % _NB
            pltpu.make_async_copy(
                bufs[b], out_hbm.at[pl.ds(0, _C)], ssem[b]
            ).wait()

        G = _NB - 1
        for j in range(G):
            start_gather(j)
        for j in range(n_chunks):
            pl.when(active(j))(lambda j=j: wait_gather(j))
            start_store(j)
            nxt = j + G
            if nxt < n_chunks:
                if j >= 1:
                    pl.when(active(nxt))(lambda j=j: wait_store(j - 1))
                start_gather(nxt)
        for j in range(n_chunks):
            nxt = j + _NB
            if nxt >= n_chunks:
                tail = active(j)
            else:
                tail = jnp.logical_and(active(j), jnp.logical_not(active(nxt)))
            pl.when(tail)(lambda j=j: wait_store(j))

    return k


def kernel(x, upsample_inds):
    D = x.shape[1]
    B = upsample_inds.shape[0]
    idx = upsample_inds[:, 0].astype(jnp.int32)
    info = plsc.get_sparse_core_info()
    NC, NS = info.num_cores, info.num_subcores
    span = NC * NS * _C
    B_pad = ((B + span - 1) // span) * span
    pad = jnp.arange(B_pad - B, dtype=jnp.int32) % x.shape[0]
    idx_pad = jnp.concatenate([idx, pad])
    return _make_gather(D, B, NC, NS)(x, idx_pad)
